# baseline (speedup 1.0000x reference)
import functools

import jax
import jax.numpy as jnp
from jax import lax
from jax.experimental import pallas as pl
from jax.experimental.pallas import tpu as pltpu
from jax.experimental.pallas import tpu_sc as plsc

S_K = 64
W_K = 128
BATCH = 16384
NC = 2
NS = 16
LANES = 16
NW = NC * NS
ROWS_PER_W = BATCH // NW
CHUNK = 128
NCHUNK = ROWS_PER_W // CHUNK

_mesh = plsc.VectorSubcoreMesh(
    core_axis_name="c", subcore_axis_name="s", num_cores=NC, num_subcores=NS
)


@functools.partial(
    pl.kernel,
    out_type=jax.ShapeDtypeStruct((BATCH,), jnp.float32),
    mesh=_mesh,
    scratch_types=[
        pltpu.VMEM((CHUNK,), jnp.int32),
        pltpu.VMEM((CHUNK,), jnp.int32),
        pltpu.VMEM((CHUNK,), jnp.int32),
        pltpu.VMEM((CHUNK, S_K), jnp.float32),
        pltpu.VMEM((CHUNK, S_K), jnp.float32),
        pltpu.VMEM((CHUNK, W_K), jnp.float32),
        pltpu.VMEM((CHUNK,), jnp.float32),
        pltpu.SemaphoreType.DMA,
    ],
    compiler_params=pltpu.CompilerParams(use_tc_tiling_on_sc=False),
)
def _sc_forward(x0, x1, x2, s_tab, r_tab, w_tab, out,
                idx0, idx1, idx2, srows, rrows, wrows, outv, sem):
    wid = lax.axis_index("s") * NC + lax.axis_index("c")
    lane = lax.iota(jnp.int32, LANES)

    _dnums = lax.GatherDimensionNumbers(
        offset_dims=(), collapsed_slice_dims=(0,), start_index_map=(0,)
    )

    def _lane_shuffle(v, idx):
        return lax.gather(
            v, idx[:, None], _dnums, slice_sizes=(1,),
            mode=lax.GatherScatterMode.PROMISE_IN_BOUNDS,
        )

    for c in range(NCHUNK):
        base = wid * ROWS_PER_W + c * CHUNK
        pltpu.sync_copy(x0.at[pl.ds(base, CHUNK)], idx0)
        pltpu.sync_copy(x1.at[pl.ds(base, CHUNK)], idx1)
        pltpu.sync_copy(x2.at[pl.ds(base, CHUNK)], idx2)
        cs = pltpu.async_copy(s_tab.at[idx0], srows, sem)
        cr = pltpu.async_copy(r_tab.at[idx1], rrows, sem)
        cw = pltpu.async_copy(w_tab.at[idx2], wrows, sem)
        cs.wait()
        cr.wait()
        cw.wait()

        def group_body(g, carry):
            def row_body(j, acc_out):
                i = g * LANES + j
                acc = srows[i, pl.ds(0, LANES)] * wrows[i, pl.ds(0, LANES)]
                for k in range(1, S_K // LANES):
                    acc = acc + (srows[i, pl.ds(k * LANES, LANES)]
                                 * wrows[i, pl.ds(k * LANES, LANES)])
                for k in range(S_K // LANES):
                    acc = acc + (rrows[i, pl.ds(k * LANES, LANES)]
                                 * wrows[i, pl.ds(S_K + k * LANES, LANES)])
                for d in (8, 4, 2, 1):
                    acc = acc + _lane_shuffle(acc, lane ^ d)
                return jnp.where(lane == j, acc, acc_out)

            accs = lax.fori_loop(
                0, LANES, row_body, jnp.zeros((LANES,), jnp.float32)
            )
            outv[pl.ds(g * LANES, LANES)] = 1.0 / (1.0 + jnp.exp(-accs))
            return carry

        lax.fori_loop(0, CHUNK // LANES, group_body, 0)
        pltpu.sync_copy(outv, out.at[pl.ds(base, CHUNK)])


def kernel(X, s_embeds, r_embeds, w_embeds):
    Xi = X.astype(jnp.int32)
    return _sc_forward(Xi[:, 0], Xi[:, 1], Xi[:, 2],
                       s_embeds, r_embeds, w_embeds)
